# baseline (speedup 1.0000x reference)
import functools

import jax
import jax.numpy as jnp
from jax import lax
from jax.experimental import pallas as pl
from jax.experimental.pallas import tpu as pltpu
from jax.experimental.pallas import tpu_sc as plsc

B = 16
N = 2048
K = 16
N_SPACE = 4
N_ALL = 16
NUM_CLASSES = 10
LAYER_DIMS = [15, 20, 25, 30, 35, 40]

_NC, _NS = 2, 16
_NW = _NC * _NS
_CH = 1024
_HB = B // 2
_TOTAL_H = _HB * N * K
_NCHUNK_H = _TOTAL_H // _NW // _CH


def _make_sc_gather(c_pad, total_rows, nchunk):
    mesh = plsc.VectorSubcoreMesh(core_axis_name="c", subcore_axis_name="s")
    rpw = total_rows // _NW

    @functools.partial(
        pl.kernel,
        out_type=jax.ShapeDtypeStruct((total_rows, c_pad), jnp.bfloat16),
        mesh=mesh,
        scratch_types=[
            pltpu.VMEM((nchunk, _CH), jnp.int32),
            pltpu.VMEM((2, _CH, c_pad), jnp.bfloat16),
            pltpu.SemaphoreType.DMA,
            pltpu.SemaphoreType.DMA,
        ],
        compiler_params=pltpu.CompilerParams(use_tc_tiling_on_sc=False),
    )
    def gather_k(table_hbm, idx_hbm, out_hbm, idx_v, rows_v, sem_g, sem_w):
        wid = lax.axis_index("s") * _NC + lax.axis_index("c")
        base = wid * rpw
        pltpu.sync_copy(idx_hbm.at[wid], idx_v)
        gathers = [None, None]
        wbs = [None] * nchunk
        gathers[0] = pltpu.async_copy(
            table_hbm.at[idx_v.at[0]], rows_v.at[0], sem_g)
        for j in range(nchunk):
            p = j % 2
            if j + 1 < nchunk:
                if j >= 1:
                    wbs[j - 1].wait()
                gathers[(j + 1) % 2] = pltpu.async_copy(
                    table_hbm.at[idx_v.at[j + 1]], rows_v.at[(j + 1) % 2],
                    sem_g)
            gathers[p].wait()
            wbs[j] = pltpu.async_copy(
                rows_v.at[p], out_hbm.at[pl.ds(base + j * _CH, _CH)], sem_w)
        wbs[nchunk - 1].wait()

    return gather_k


def _mm_kernel(pre_ref, w_ref, b_ref, out_ref):
    pre = pre_ref[0]
    acc = jnp.dot(pre, w_ref[...], preferred_element_type=jnp.float32)
    out_ref[0] = jnp.maximum(acc + b_ref[...], 0.0).astype(out_ref.dtype)


def _layer_matmul(pre, W, b):
    Bb, n, kc = pre.shape
    d = W.shape[1]
    return pl.pallas_call(
        _mm_kernel,
        grid=(Bb,),
        in_specs=[
            pl.BlockSpec((1, n, kc), lambda i: (i, 0, 0)),
            pl.BlockSpec((kc, d), lambda i: (0, 0)),
            pl.BlockSpec((1, d), lambda i: (0, 0)),
        ],
        out_specs=pl.BlockSpec((1, n, d), lambda i: (i, 0, 0)),
        out_shape=jax.ShapeDtypeStruct((Bb, n, d), jnp.bfloat16),
    )(pre, W, b)


def _head_kernel(feat_ref, ne_ref, w1_ref, b1_ref, w2_ref, b2_ref,
                 w3_ref, b3_ref, out_ref):
    feat = feat_ref[...].astype(jnp.float32)
    bb, n, _ = feat_ref.shape
    sq = ne_ref[...]
    pos = jax.lax.broadcasted_iota(jnp.int32, (bb, n), 1)
    mask = (pos < sq).astype(jnp.float32)
    s = jnp.sum(feat * mask[..., None], axis=1)
    flattened = jnp.clip(s / sq.astype(jnp.float32), -1e9, 1e9)
    h1 = jnp.maximum(jnp.dot(flattened, w1_ref[...],
                             preferred_element_type=jnp.float32)
                     + b1_ref[...], 0.0)
    h2 = jnp.maximum(jnp.dot(h1, w2_ref[...],
                             preferred_element_type=jnp.float32)
                     + b2_ref[...], 0.0)
    out_ref[...] = jnp.dot(h2, w3_ref[...],
                           preferred_element_type=jnp.float32) + b3_ref[...]


def _head(feat, num_entries, w1, b1, w2, b2, w3, b3):
    Bb = feat.shape[0]
    return pl.pallas_call(
        _head_kernel,
        out_shape=jax.ShapeDtypeStruct((Bb, NUM_CLASSES), jnp.float32),
    )(feat, num_entries, w1, b1, w2, b2, w3, b3)


def _pad8(c):
    return -(-c // 8) * 8


def _pad_weight(W, c_in, c_pad, w_out, w_pad):
    W3 = W.reshape(K, c_in, w_out)
    W3 = jnp.pad(W3, ((0, 0), (0, c_pad - c_in), (0, w_pad - w_out)))
    return W3.reshape(K * c_pad, w_pad)


def kernel(space_features, all_features, neighbors_matrix, num_entries, params):
    nbr = neighbors_matrix.astype(jnp.int32)
    offs = (jnp.arange(_HB, dtype=jnp.int32) * N)[:, None, None]
    idx_h = [
        (nbr[h * _HB:(h + 1) * _HB] + offs).reshape(_NW, _NCHUNK_H, _CH)
        for h in range(2)
    ]

    c_ins = [N_SPACE + N_ALL] + [N_SPACE + d for d in LAYER_DIMS[:-1]]
    c_pads = [_pad8(c) for c in c_ins]
    w_outs = [N_SPACE + d for d in LAYER_DIMS]
    w_pads = c_pads[1:] + [_pad8(w_outs[-1])]

    cat = jnp.concatenate([space_features, all_features], axis=-1)
    cat = jnp.pad(cat, ((0, 0), (0, 0), (0, c_pads[0] - c_ins[0])))
    cat = cat.astype(jnp.bfloat16)
    cat_h = [cat[:_HB], cat[_HB:]]

    for i in range(len(LAYER_DIMS)):
        Wcat = jnp.concatenate([params["Ws%d" % i], params["Wa%d" % i]],
                               axis=1)
        bcat = jnp.concatenate([params["bs%d" % i], params["ba%d" % i]])
        Wp = _pad_weight(Wcat, c_ins[i], c_pads[i], w_outs[i],
                         w_pads[i]).astype(jnp.bfloat16)
        bp = jnp.pad(bcat, (0, w_pads[i] - w_outs[i]))[None]

        gather = _make_sc_gather(c_pads[i], _TOTAL_H, _NCHUNK_H)
        pre_h = [gather(cat_h[h].reshape(_HB * N, c_pads[i]), idx_h[h])
                 for h in range(2)]
        cat_h = [
            _layer_matmul(pre_h[h].reshape(_HB, N, K * c_pads[i]), Wp, bp)
            for h in range(2)
        ]

    fc_args = (params["W_fc1"], params["b_fc1"][None],
               params["W_fc2"], params["b_fc2"][None],
               params["W_fc3"], params["b_fc3"][None])
    logits_h = [
        _head(cat_h[h][:, :, N_SPACE:N_SPACE + LAYER_DIMS[-1]],
              num_entries[h * _HB:(h + 1) * _HB], *fc_args)
        for h in range(2)
    ]
    return jnp.concatenate(logits_h, axis=0)


# speedup vs baseline: 2.8907x; 2.8907x over previous
import functools

import jax
import jax.numpy as jnp
from jax import lax
from jax.experimental import pallas as pl
from jax.experimental.pallas import tpu as pltpu
from jax.experimental.pallas import tpu_sc as plsc

B = 16
N = 2048
K = 16
N_SPACE = 4
N_ALL = 16
NUM_CLASSES = 10
LAYER_DIMS = [15, 20, 25, 30, 35, 40]

_NC, _NS = 2, 16
_NW = _NC * _NS
_CH = 1024
_HB = B // 2
_TOTAL_H = _HB * N * K
_NCHUNK_H = _TOTAL_H // _NW // _CH


def _make_sc_gather(c_pad, total_rows, nchunk):
    mesh = plsc.VectorSubcoreMesh(core_axis_name="c", subcore_axis_name="s")
    rpw = total_rows // _NW

    @functools.partial(
        pl.kernel,
        out_type=jax.ShapeDtypeStruct((total_rows, c_pad), jnp.float32),
        mesh=mesh,
        scratch_types=[
            pltpu.VMEM((nchunk, _CH), jnp.int32),
            pltpu.VMEM((2, _CH, c_pad), jnp.float32),
            pltpu.SemaphoreType.DMA,
            pltpu.SemaphoreType.DMA,
        ],
        compiler_params=pltpu.CompilerParams(use_tc_tiling_on_sc=False),
    )
    def gather_k(table_hbm, idx_hbm, out_hbm, idx_v, rows_v, sem_g, sem_w):
        wid = lax.axis_index("s") * _NC + lax.axis_index("c")
        base = wid * rpw
        pltpu.sync_copy(idx_hbm.at[wid], idx_v)
        gathers = [None, None]
        wbs = [None] * nchunk
        gathers[0] = pltpu.async_copy(
            table_hbm.at[idx_v.at[0]], rows_v.at[0], sem_g)
        for j in range(nchunk):
            p = j % 2
            if j + 1 < nchunk:
                if j >= 1:
                    wbs[j - 1].wait()
                gathers[(j + 1) % 2] = pltpu.async_copy(
                    table_hbm.at[idx_v.at[j + 1]], rows_v.at[(j + 1) % 2],
                    sem_g)
            gathers[p].wait()
            wbs[j] = pltpu.async_copy(
                rows_v.at[p], out_hbm.at[pl.ds(base + j * _CH, _CH)], sem_w)
        wbs[nchunk - 1].wait()

    return gather_k


def _mm_kernel(pre_ref, w_ref, b_ref, out_ref):
    pre = pre_ref[0]
    acc = jnp.dot(pre, w_ref[...], preferred_element_type=jnp.float32)
    out_ref[0] = jnp.maximum(acc + b_ref[...], 0.0)


def _layer_matmul(pre, W, b):
    Bb, n, kc = pre.shape
    d = W.shape[1]
    return pl.pallas_call(
        _mm_kernel,
        grid=(Bb,),
        in_specs=[
            pl.BlockSpec((1, n, kc), lambda i: (i, 0, 0)),
            pl.BlockSpec((kc, d), lambda i: (0, 0)),
            pl.BlockSpec((1, d), lambda i: (0, 0)),
        ],
        out_specs=pl.BlockSpec((1, n, d), lambda i: (i, 0, 0)),
        out_shape=jax.ShapeDtypeStruct((Bb, n, d), jnp.float32),
    )(pre, W, b)


def _head_kernel(feat_ref, ne_ref, w1_ref, b1_ref, w2_ref, b2_ref,
                 w3_ref, b3_ref, out_ref):
    feat = feat_ref[...]
    bb, n, _ = feat_ref.shape
    sq = ne_ref[...]
    pos = jax.lax.broadcasted_iota(jnp.int32, (bb, n), 1)
    mask = (pos < sq).astype(jnp.float32)
    s = jnp.sum(feat * mask[..., None], axis=1)
    flattened = jnp.clip(s / sq.astype(jnp.float32), -1e9, 1e9)
    h1 = jnp.maximum(jnp.dot(flattened, w1_ref[...],
                             preferred_element_type=jnp.float32)
                     + b1_ref[...], 0.0)
    h2 = jnp.maximum(jnp.dot(h1, w2_ref[...],
                             preferred_element_type=jnp.float32)
                     + b2_ref[...], 0.0)
    out_ref[...] = jnp.dot(h2, w3_ref[...],
                           preferred_element_type=jnp.float32) + b3_ref[...]


def _head(feat, num_entries, w1, b1, w2, b2, w3, b3):
    Bb = feat.shape[0]
    return pl.pallas_call(
        _head_kernel,
        out_shape=jax.ShapeDtypeStruct((Bb, NUM_CLASSES), jnp.float32),
    )(feat, num_entries, w1, b1, w2, b2, w3, b3)


def _pad8(c):
    return -(-c // 8) * 8


def _pad_weight(W, c_in, c_pad, w_out, w_pad):
    W3 = W.reshape(K, c_in, w_out)
    W3 = jnp.pad(W3, ((0, 0), (0, c_pad - c_in), (0, w_pad - w_out)))
    return W3.reshape(K * c_pad, w_pad)


def kernel(space_features, all_features, neighbors_matrix, num_entries, params):
    nbr = neighbors_matrix.astype(jnp.int32)
    offs = (jnp.arange(_HB, dtype=jnp.int32) * N)[:, None, None]
    idx_h = [
        (nbr[h * _HB:(h + 1) * _HB] + offs).reshape(_NW, _NCHUNK_H, _CH)
        for h in range(2)
    ]

    c_ins = [N_SPACE + N_ALL] + [N_SPACE + d for d in LAYER_DIMS[:-1]]
    c_pads = [_pad8(c) for c in c_ins]
    w_outs = [N_SPACE + d for d in LAYER_DIMS]
    w_pads = c_pads[1:] + [_pad8(w_outs[-1])]

    cat = jnp.concatenate([space_features, all_features], axis=-1)
    cat = jnp.pad(cat, ((0, 0), (0, 0), (0, c_pads[0] - c_ins[0])))
    cat_h = [cat[:_HB], cat[_HB:]]

    for i in range(len(LAYER_DIMS)):
        Wcat = jnp.concatenate([params["Ws%d" % i], params["Wa%d" % i]],
                               axis=1)
        bcat = jnp.concatenate([params["bs%d" % i], params["ba%d" % i]])
        Wp = _pad_weight(Wcat, c_ins[i], c_pads[i], w_outs[i], w_pads[i])
        bp = jnp.pad(bcat, (0, w_pads[i] - w_outs[i]))[None]

        gather = _make_sc_gather(c_pads[i], _TOTAL_H, _NCHUNK_H)
        pre_h = [gather(cat_h[h].reshape(_HB * N, c_pads[i]), idx_h[h])
                 for h in range(2)]
        cat_h = [
            _layer_matmul(pre_h[h].reshape(_HB, N, K * c_pads[i]), Wp, bp)
            for h in range(2)
        ]

    fc_args = (params["W_fc1"], params["b_fc1"][None],
               params["W_fc2"], params["b_fc2"][None],
               params["W_fc3"], params["b_fc3"][None])
    logits_h = [
        _head(cat_h[h][:, :, N_SPACE:N_SPACE + LAYER_DIMS[-1]],
              num_entries[h * _HB:(h + 1) * _HB], *fc_args)
        for h in range(2)
    ]
    return jnp.concatenate(logits_h, axis=0)
